# baseline (speedup 1.0000x reference)
import functools

import jax
import jax.numpy as jnp
from jax import lax
from jax.experimental import pallas as pl
from jax.experimental.pallas import tpu as pltpu
from jax.experimental.pallas import tpu_sc as plsc

_M = 10000
_F = 128
_E = 320000
_NW = 32
_EPW = _E // _NW
_C = 80
_NCHUNK = _EPW // _C
_RC = 200
_NRC = _M // _RC
_RREP = -(-_NRC // 16)
_LG = _F // 16


def _spmv_body(h_hbm, ed_hbm, out_hbm, *refs):
    ed = refs[0:8]
    rows = refs[8:12]
    acc_sh = refs[12]
    sem_g = refs[13:17]
    sem_s = refs[17:21]
    sem_e = refs[21:29]
    c = lax.axis_index("c")
    s = lax.axis_index("s")
    wid = c * 16 + s

    r0 = rows[0]

    def _zrow(r, carry):
        for j in range(_LG):
            r0[r, pl.ds(j * 16, 16)] = jnp.zeros((16,), jnp.float32)
        return carry
    lax.fori_loop(0, _C, _zrow, 0)
    for rep in range(-(-_NCHUNK // 16)):
        cid = s + 16 * rep

        @pl.when(cid < _NCHUNK)
        def _():
            pltpu.sync_copy(r0, acc_sh.at[pl.ds(cid * _C, _C)])
    plsc.subcore_barrier()

    def _fire_edata(t, e):
        pltpu.async_copy(ed_hbm.at[wid, t], ed[e], sem_e[e])

    def _wait_edata(t, e):
        pltpu.make_async_copy(ed_hbm.at[wid, t], ed[e], sem_e[e]).wait()

    def _gather(t, b, e):
        pltpu.async_copy(h_hbm.at[ed[e].at[0]], rows[b], sem_g[b])

    def _wait_gather(t, b, e):
        pltpu.make_async_copy(h_hbm.at[ed[e].at[0]], rows[b], sem_g[b]).wait()

    def _scatter(t, b, e):
        pltpu.async_copy(rows[b], acc_sh.at[ed[e].at[1]], sem_s[b], add=True)

    def _wait_scatter(t, b, e):
        pltpu.make_async_copy(rows[b], acc_sh.at[ed[e].at[1]], sem_s[b]).wait()

    def _scale(b, e):
        rb = rows[b]
        eb = ed[e]

        @plsc.parallel_loop(0, _C // 16, unroll=2)
        def _e16(g):
            w16 = lax.bitcast_convert_type(eb[2, pl.ds(g * 16, 16)], jnp.float32)
            for i in range(16):
                wv = lax.gather(
                    w16, jnp.full((16, 1), i, jnp.int32),
                    dimension_numbers=lax.GatherDimensionNumbers(
                        offset_dims=(), collapsed_slice_dims=(0,),
                        start_index_map=(0,)),
                    slice_sizes=(1,),
                    mode=lax.GatherScatterMode.PROMISE_IN_BOUNDS)
                for j in range(_LG):
                    sl = pl.ds(j * 16, 16)
                    rb[g * 16 + i, sl] = rb[g * 16 + i, sl] * wv

    for t in range(6):
        _fire_edata(t, t)
    _wait_edata(0, 0)
    _gather(0, 0, 0)
    _wait_edata(1, 1)
    _gather(1, 1, 1)

    def _substep(t, b, e):
        bf = (b + 2) % 4

        @pl.when(t + 2 < _NCHUNK)
        def _():
            @pl.when(t >= 2)
            def _():
                _wait_scatter(t - 2, bf, (e + 6) % 8)

            @pl.when(t + 6 < _NCHUNK)
            def _():
                _fire_edata(t + 6, (e + 6) % 8)
            _wait_edata(t + 2, (e + 2) % 8)
            _gather(t + 2, bf, (e + 2) % 8)
        _wait_gather(t, b, e)
        _scale(b, e)
        _scatter(t, b, e)

    def _outer(i, carry):
        for b8 in range(8):
            _substep(i * 8 + b8, b8 % 4, b8)
        return carry
    lax.fori_loop(0, _NCHUNK // 8, _outer, 0)

    for t in range((_NCHUNK // 8) * 8, _NCHUNK):
        _substep(t, t % 4, t % 8)
    for t in range(_NCHUNK - 4, _NCHUNK):
        _wait_scatter(t, t % 4, t % 8)

    plsc.subcore_barrier()
    for rep in range(-(-_NCHUNK // 16)):
        cid = s + 16 * rep

        @pl.when(cid < _NCHUNK)
        def _():
            o = cid * _C
            pltpu.sync_copy(acc_sh.at[pl.ds(o, _C)], out_hbm.at[c, pl.ds(o, _C)])


@functools.lru_cache(maxsize=None)
def _make_spmv():
    scratch = (
        [pltpu.VMEM((3, _C), jnp.int32) for _ in range(8)]
        + [pltpu.VMEM((_C, _F), jnp.float32) for _ in range(4)]
        + [pltpu.VMEM_SHARED((_M, _F), jnp.float32)]
        + [pltpu.SemaphoreType.DMA] * 16
    )
    return pl.kernel(
        _spmv_body,
        out_type=jax.ShapeDtypeStruct((2, _M, _F), jnp.float32),
        scratch_types=scratch,
        mesh=plsc.VectorSubcoreMesh(core_axis_name="c", subcore_axis_name="s"),
    )


def _combine_body(p_ref, o_ref):
    o_ref[...] = p_ref[0] + p_ref[1]


def _combine(p):
    return pl.pallas_call(
        _combine_body,
        grid=(10,),
        in_specs=[pl.BlockSpec((2, _M // 10, _F), lambda i: (0, i, 0))],
        out_specs=pl.BlockSpec((_M // 10, _F), lambda i: (i, 0)),
        out_shape=jax.ShapeDtypeStruct((_M, _F), jnp.float32),
    )(p)


def _dense_body(x0_ref, s1_ref, s2_ref, p3_ref, dk_ref, pk_ref, o_ref):
    dk = dk_ref[...]
    pkt = pk_ref[...].T
    w0 = pkt * dk[:, 0, 0][:, None]
    w1 = pkt * dk[:, 0, 1][:, None]
    w2 = pkt * dk[:, 0, 2][:, None]
    w3 = pkt * dk[:, 0, 3][:, None]
    v0 = w0 - w2
    v1 = w1 - 3.0 * w3
    v2 = 2.0 * w2
    v3 = 4.0 * w3
    s3 = p3_ref[0] + p3_ref[1]
    acc = jnp.dot(x0_ref[...], v0, preferred_element_type=jnp.float32)
    acc += jnp.dot(s1_ref[...], v1, preferred_element_type=jnp.float32)
    acc += jnp.dot(s2_ref[...], v2, preferred_element_type=jnp.float32)
    acc += jnp.dot(s3, v3, preferred_element_type=jnp.float32)
    o_ref[...] = acc


def _dense(x0, s1, s2, p3, dkernel, pkernel):
    bm = _M // 10
    return pl.pallas_call(
        _dense_body,
        grid=(10,),
        in_specs=[
            pl.BlockSpec((bm, _F), lambda i: (i, 0)),
            pl.BlockSpec((bm, _F), lambda i: (i, 0)),
            pl.BlockSpec((bm, _F), lambda i: (i, 0)),
            pl.BlockSpec((2, bm, _F), lambda i: (0, i, 0)),
            pl.BlockSpec((_F, 1, 4), lambda i: (0, 0, 0)),
            pl.BlockSpec((_F, _F), lambda i: (0, 0)),
        ],
        out_specs=pl.BlockSpec((bm, _F), lambda i: (i, 0)),
        out_shape=jax.ShapeDtypeStruct((_M, _F), jnp.float32),
    )(x0, s1, s2, p3, dkernel, pkernel)


def kernel(x, edge_index, edge_weight, dkernel, pkernel):
    m = x.shape[1]
    x0 = jnp.transpose(x, (1, 0, 2)).reshape(m, -1)
    src = edge_index[0].astype(jnp.int32).reshape(_NW, _NCHUNK, _C)
    dst = edge_index[1].astype(jnp.int32).reshape(_NW, _NCHUNK, _C)
    wbits = lax.bitcast_convert_type(
        edge_weight.astype(jnp.float32), jnp.int32).reshape(_NW, _NCHUNK, _C)
    ed = jnp.stack([src, dst, wbits], axis=2)

    spmv = _make_spmv()
    p1 = spmv(x0, ed)
    s1 = _combine(p1)
    p2 = spmv(s1, ed)
    s2 = _combine(p2)
    p3 = spmv(s2, ed)

    out = _dense(x0, s1, s2, p3, dkernel, pkernel)
    return out.reshape(1, m, -1)


# speedup vs baseline: 1.1358x; 1.1358x over previous
import functools

import jax
import jax.numpy as jnp
from jax import lax
from jax.experimental import pallas as pl
from jax.experimental.pallas import tpu as pltpu
from jax.experimental.pallas import tpu_sc as plsc

_M = 10000
_F = 128
_E = 320000
_NW = 32
_EPW = _E // _NW
_C = 80
_NCHUNK = _EPW // _C
_RC = 200
_NRC = _M // _RC
_RREP = -(-_NRC // 16)
_LG = _F // 16


def _spmv_body(h_hbm, ed_hbm, out_hbm, *refs):
    ed = refs[0:8]
    rows = refs[8:12]
    acc_sh = refs[12]
    sem_g = refs[13:17]
    sem_s = refs[17:21]
    sem_e = refs[21:29]
    c = lax.axis_index("c")
    s = lax.axis_index("s")
    wid = c * 16 + s

    def _fire_edata(t, e):
        pltpu.async_copy(ed_hbm.at[wid, t], ed[e], sem_e[e])

    def _wait_edata(t, e):
        pltpu.make_async_copy(ed_hbm.at[wid, t], ed[e], sem_e[e]).wait()

    def _gather(t, b, e):
        pltpu.async_copy(h_hbm.at[ed[e].at[0]], rows[b], sem_g[b])

    def _wait_gather(t, b, e):
        pltpu.make_async_copy(h_hbm.at[ed[e].at[0]], rows[b], sem_g[b]).wait()

    def _scatter(t, b, e):
        pltpu.async_copy(rows[b], acc_sh.at[ed[e].at[1]], sem_s[b], add=True)

    def _wait_scatter(t, b, e):
        pltpu.make_async_copy(rows[b], acc_sh.at[ed[e].at[1]], sem_s[b]).wait()

    def _scale(b, e):
        rb = rows[b]
        eb = ed[e]

        def _e16(g, carry):
            w16 = lax.bitcast_convert_type(eb[2, pl.ds(g * 16, 16)], jnp.float32)
            for i in range(16):
                wv = lax.gather(
                    w16, jnp.full((16, 1), i, jnp.int32),
                    dimension_numbers=lax.GatherDimensionNumbers(
                        offset_dims=(), collapsed_slice_dims=(0,),
                        start_index_map=(0,)),
                    slice_sizes=(1,),
                    mode=lax.GatherScatterMode.PROMISE_IN_BOUNDS)
                for j in range(_LG):
                    sl = pl.ds(j * 16, 16)
                    rb[g * 16 + i, sl] = rb[g * 16 + i, sl] * wv
            return carry
        lax.fori_loop(0, _C // 16, _e16, 0)

    for t in range(6):
        _fire_edata(t, t)

    r3 = rows[3]

    def _zrow(r, carry):
        for j in range(_LG):
            r3[r, pl.ds(j * 16, 16)] = jnp.zeros((16,), jnp.float32)
        return carry
    lax.fori_loop(0, _C, _zrow, 0)
    for rep in range(-(-_NCHUNK // 16)):
        cid = s + 16 * rep

        @pl.when(cid < _NCHUNK)
        def _():
            pltpu.sync_copy(r3, acc_sh.at[pl.ds(cid * _C, _C)])
    _wait_edata(0, 0)
    _gather(0, 0, 0)
    _wait_edata(1, 1)
    _gather(1, 1, 1)
    plsc.subcore_barrier()

    def _substep(t, b, e):
        bf = (b + 2) % 4

        @pl.when(t + 2 < _NCHUNK)
        def _():
            @pl.when(t >= 2)
            def _():
                _wait_scatter(t - 2, bf, (e + 6) % 8)

            @pl.when(t + 6 < _NCHUNK)
            def _():
                _fire_edata(t + 6, (e + 6) % 8)
            _wait_edata(t + 2, (e + 2) % 8)
            _gather(t + 2, bf, (e + 2) % 8)
        _wait_gather(t, b, e)
        _scale(b, e)
        _scatter(t, b, e)

    def _outer(i, carry):
        for b8 in range(8):
            _substep(i * 8 + b8, b8 % 4, b8)
        return carry
    lax.fori_loop(0, _NCHUNK // 8, _outer, 0)

    for t in range((_NCHUNK // 8) * 8, _NCHUNK):
        _substep(t, t % 4, t % 8)
    for t in range(_NCHUNK - 4, _NCHUNK):
        _wait_scatter(t, t % 4, t % 8)

    plsc.subcore_barrier()
    for rep in range(-(-_NCHUNK // 16)):
        cid = s + 16 * rep

        @pl.when(cid < _NCHUNK)
        def _():
            o = cid * _C
            pltpu.sync_copy(acc_sh.at[pl.ds(o, _C)], out_hbm.at[c, pl.ds(o, _C)])


@functools.lru_cache(maxsize=None)
def _make_spmv():
    scratch = (
        [pltpu.VMEM((3, _C), jnp.int32) for _ in range(8)]
        + [pltpu.VMEM((_C, _F), jnp.float32) for _ in range(4)]
        + [pltpu.VMEM_SHARED((_M, _F), jnp.float32)]
        + [pltpu.SemaphoreType.DMA] * 16
    )
    return pl.kernel(
        _spmv_body,
        out_type=jax.ShapeDtypeStruct((2, _M, _F), jnp.float32),
        scratch_types=scratch,
        mesh=plsc.VectorSubcoreMesh(core_axis_name="c", subcore_axis_name="s"),
    )


def _combine_body(p_ref, o_ref):
    o_ref[...] = p_ref[0] + p_ref[1]


def _combine(p):
    return pl.pallas_call(
        _combine_body,
        grid=(10,),
        in_specs=[pl.BlockSpec((2, _M // 10, _F), lambda i: (0, i, 0))],
        out_specs=pl.BlockSpec((_M // 10, _F), lambda i: (i, 0)),
        out_shape=jax.ShapeDtypeStruct((_M, _F), jnp.float32),
    )(p)


def _dense_body(x0_ref, s1_ref, s2_ref, p3_ref, dk_ref, pk_ref, o_ref):
    dk = dk_ref[...]
    pkt = pk_ref[...].T
    w0 = pkt * dk[:, 0, 0][:, None]
    w1 = pkt * dk[:, 0, 1][:, None]
    w2 = pkt * dk[:, 0, 2][:, None]
    w3 = pkt * dk[:, 0, 3][:, None]
    v0 = w0 - w2
    v1 = w1 - 3.0 * w3
    v2 = 2.0 * w2
    v3 = 4.0 * w3
    s3 = p3_ref[0] + p3_ref[1]
    acc = jnp.dot(x0_ref[...], v0, preferred_element_type=jnp.float32)
    acc += jnp.dot(s1_ref[...], v1, preferred_element_type=jnp.float32)
    acc += jnp.dot(s2_ref[...], v2, preferred_element_type=jnp.float32)
    acc += jnp.dot(s3, v3, preferred_element_type=jnp.float32)
    o_ref[...] = acc


def _dense(x0, s1, s2, p3, dkernel, pkernel):
    bm = _M // 10
    return pl.pallas_call(
        _dense_body,
        grid=(10,),
        in_specs=[
            pl.BlockSpec((bm, _F), lambda i: (i, 0)),
            pl.BlockSpec((bm, _F), lambda i: (i, 0)),
            pl.BlockSpec((bm, _F), lambda i: (i, 0)),
            pl.BlockSpec((2, bm, _F), lambda i: (0, i, 0)),
            pl.BlockSpec((_F, 1, 4), lambda i: (0, 0, 0)),
            pl.BlockSpec((_F, _F), lambda i: (0, 0)),
        ],
        out_specs=pl.BlockSpec((bm, _F), lambda i: (i, 0)),
        out_shape=jax.ShapeDtypeStruct((_M, _F), jnp.float32),
    )(x0, s1, s2, p3, dkernel, pkernel)


def kernel(x, edge_index, edge_weight, dkernel, pkernel):
    m = x.shape[1]
    x0 = jnp.transpose(x, (1, 0, 2)).reshape(m, -1)
    src = edge_index[0].astype(jnp.int32).reshape(_NW, _NCHUNK, _C)
    dst = edge_index[1].astype(jnp.int32).reshape(_NW, _NCHUNK, _C)
    wbits = lax.bitcast_convert_type(
        edge_weight.astype(jnp.float32), jnp.int32).reshape(_NW, _NCHUNK, _C)
    ed = jnp.stack([src, dst, wbits], axis=2)

    spmv = _make_spmv()
    p1 = spmv(x0, ed)
    s1 = _combine(p1)
    p2 = spmv(s1, ed)
    s2 = _combine(p2)
    p3 = spmv(s2, ed)

    out = _dense(x0, s1, s2, p3, dkernel, pkernel)
    return out.reshape(1, m, -1)
